# baseline (speedup 1.0000x reference)
import functools

import jax
import jax.numpy as jnp
from jax.experimental import pallas as pl


_EPS = 1e-5
_OFFS = tuple(range(1, 11)) + tuple(-k for k in range(1, 11))


def _lap_seq(z, V):
    zp = jnp.concatenate([z[:, V - 10:, :], z, z[:, :10, :]], axis=1)
    acc = jnp.zeros_like(z)
    w = -1.0 / 20.0
    for off in _OFFS:
        acc = acc + zp[:, 10 + off:10 + off + V, :] * w
    return acc


def _mm(z, wk):
    m = z.shape[0] * z.shape[1]
    return jax.lax.dot_general(
        z.reshape(m, z.shape[2]), wk, (((1,), (0,)), ((), ())),
        preferred_element_type=jnp.float32,
        precision=jax.lax.Precision.DEFAULT).reshape(
            z.shape[0], z.shape[1], -1)


def _affine_relu(y, m, d, g, be):
    return jnp.maximum((y - m) / d * g + be, 0.0)


def _cheb_body(y_ref, m_ref, d_ref, g_ref, be_ref, w_ref, b_ref, out_ref,
               outT_ref, *, act, V):
    y = y_ref[...]
    if act:
        x = _affine_relu(y, m_ref[0], d_ref[0], g_ref[0], be_ref[0])
    else:
        x = y
    t1 = _lap_seq(x, V)
    t2 = 2.0 * _lap_seq(t1, V) - x
    out_ref[...] = _mm(x, w_ref[0])
    out_ref[...] += _mm(t1, w_ref[1])
    out_ref[...] += _mm(t2, w_ref[2])
    out_ref[...] += b_ref[0]
    if outT_ref is not None:
        outT_ref[...] = jnp.swapaxes(out_ref[...], 1, 2)


def _cheb_block(y_in, W, b, norm=None, bt=2, emit_t=True):
    B, V, Cin = y_in.shape
    K, _, Cout = W.shape
    act = norm is not None
    if norm is None:
        z = jnp.zeros((1, Cin), jnp.float32)
        norm = (z, z + 1.0, z + 1.0, z)
    m, dd, g, be = norm
    if emit_t:
        body = functools.partial(_cheb_body, act=act, V=V)
        out_specs = [pl.BlockSpec((bt, V, Cout), lambda i: (i, 0, 0)),
                     pl.BlockSpec((bt, Cout, V), lambda i: (i, 0, 0))]
        out_shape = [jax.ShapeDtypeStruct((B, V, Cout), jnp.float32),
                     jax.ShapeDtypeStruct((B, Cout, V), jnp.float32)]
    else:
        def body(y_ref, m_ref, d_ref, g_ref, be_ref, w_ref, b_ref, out_ref):
            _cheb_body(y_ref, m_ref, d_ref, g_ref, be_ref, w_ref, b_ref,
                       out_ref, None, act=act, V=V)
        out_specs = pl.BlockSpec((bt, V, Cout), lambda i: (i, 0, 0))
        out_shape = jax.ShapeDtypeStruct((B, V, Cout), jnp.float32)
    vec = pl.BlockSpec((1, Cin), lambda i: (0, 0))
    res = pl.pallas_call(
        body,
        grid=(B // bt,),
        in_specs=[
            pl.BlockSpec((bt, V, Cin), lambda i: (i, 0, 0)),
            vec, vec, vec, vec,
            pl.BlockSpec((K, Cin, Cout), lambda i: (0, 0, 0)),
            pl.BlockSpec((1, Cout), lambda i: (0, 0)),
        ],
        out_specs=out_specs,
        out_shape=out_shape,
    )(y_in, m, dd, g, be, W, b.reshape(1, Cout))
    return res


def _stats(yT, pr):
    yv = jnp.transpose(yT, (0, 2, 1))
    mean = jnp.mean(yv, axis=(0, 1))
    den = jnp.sqrt(jnp.var(yv, axis=(0, 1)) + _EPS)
    return (mean.reshape(1, -1), den.reshape(1, -1),
            pr['g'].reshape(1, -1), pr['be'].reshape(1, -1))


def _pool_body(y_ref, m_ref, d_ref, g_ref, be_ref, out_ref, *, C):
    y = y_ref[...]
    xs = [_affine_relu(y[:, :, j * C:(j + 1) * C], m_ref[0], d_ref[0],
                       g_ref[0], be_ref[0]) for j in range(4)]
    out_ref[...] = jnp.maximum(jnp.maximum(xs[0], xs[1]),
                               jnp.maximum(xs[2], xs[3]))


def _pool(y, norm, bt=4):
    B, V, C = y.shape
    G = V // 4
    yg = y.reshape(B, G, 4 * C)
    m, dd, g, be = norm
    body = functools.partial(_pool_body, C=C)
    vec = pl.BlockSpec((1, C), lambda i: (0, 0))
    return pl.pallas_call(
        body,
        grid=(B // bt,),
        in_specs=[pl.BlockSpec((bt, G, 4 * C), lambda i: (i, 0, 0)),
                  vec, vec, vec, vec],
        out_specs=pl.BlockSpec((bt, G, C), lambda i: (i, 0, 0)),
        out_shape=jax.ShapeDtypeStruct((B, G, C), jnp.float32),
    )(yg, m, dd, g, be)


def _unpool_body(yl_ref, ml_ref, dl_ref, gl_ref, bl_ref,
                 ys_ref, ms_ref, ds_ref, gs_ref, bs_ref, out_ref, *, C):
    xl = _affine_relu(yl_ref[...], ml_ref[0], dl_ref[0], gl_ref[0], bl_ref[0])
    ys = ys_ref[...]
    xs = [_affine_relu(ys[:, :, j * C:(j + 1) * C], ms_ref[0], ds_ref[0],
                       gs_ref[0], bs_ref[0]) for j in range(4)]
    m = jnp.maximum(jnp.maximum(xs[0], xs[1]), jnp.maximum(xs[2], xs[3]))
    taken = jnp.zeros(xs[0].shape, jnp.bool_)
    for j in range(4):
        e = jnp.logical_and(xs[j] == m, jnp.logical_not(taken))
        taken = jnp.logical_or(taken, e)
        off = j * 2 * C
        out_ref[:, :, off:off + C] = jnp.where(e, xl, 0.0)
        out_ref[:, :, off + C:off + 2 * C] = xs[j]


def _unpool_concat(y_low, norm_low, y_skip, norm_skip, bt=4):
    B, G, C = y_low.shape
    ysg = y_skip.reshape(B, G, 4 * C)
    body = functools.partial(_unpool_body, C=C)
    vec = pl.BlockSpec((1, C), lambda i: (0, 0))
    out = pl.pallas_call(
        body,
        grid=(B // bt,),
        in_specs=[pl.BlockSpec((bt, G, C), lambda i: (i, 0, 0)),
                  vec, vec, vec, vec,
                  pl.BlockSpec((bt, G, 4 * C), lambda i: (i, 0, 0)),
                  vec, vec, vec, vec],
        out_specs=pl.BlockSpec((bt, G, 8 * C), lambda i: (i, 0, 0)),
        out_shape=jax.ShapeDtypeStruct((B, G, 8 * C), jnp.float32),
    )(y_low, *norm_low, ysg, *norm_skip)
    return out.reshape(B, 4 * G, 2 * C)


def kernel(x, params):
    p = params

    y11, y11t = _cheb_block(x, p['c11']['W'], p['c11']['b'], bt=4)
    n11 = _stats(y11t, p['c11'])
    y12, y12t = _cheb_block(y11, p['c12']['W'], p['c12']['b'], norm=n11, bt=4)
    n12 = _stats(y12t, p['c12'])
    y13, y13t = _cheb_block(y12, p['c13']['W'], p['c13']['b'], norm=n12, bt=4)
    n13 = _stats(y13t, p['c13'])

    x2in = _pool(y13, n13, bt=4)
    y21, y21t = _cheb_block(x2in, p['c21']['W'], p['c21']['b'], bt=8)
    n21 = _stats(y21t, p['c21'])
    y22, y22t = _cheb_block(y21, p['c22']['W'], p['c22']['b'], norm=n21, bt=8)
    n22 = _stats(y22t, p['c22'])
    y23, y23t = _cheb_block(y22, p['c23']['W'], p['c23']['b'], norm=n22, bt=8)
    n23 = _stats(y23t, p['c23'])

    x3in = _pool(y23, n23, bt=8)
    y31, y31t = _cheb_block(x3in, p['c31']['W'], p['c31']['b'], bt=16)
    n31 = _stats(y31t, p['c31'])
    y32, y32t = _cheb_block(y31, p['c32']['W'], p['c32']['b'], norm=n31, bt=16)
    n32 = _stats(y32t, p['c32'])
    y33, y33t = _cheb_block(y32, p['c33']['W'], p['c33']['b'], norm=n32, bt=16)
    n33 = _stats(y33t, p['c33'])

    u2 = _unpool_concat(y33, n33, y23, n23, bt=8)
    yd21, yd21t = _cheb_block(u2, p['d21']['W'], p['d21']['b'], bt=8)
    nd21 = _stats(yd21t, p['d21'])
    yd22, yd22t = _cheb_block(yd21, p['d22']['W'], p['d22']['b'], norm=nd21, bt=8)
    nd22 = _stats(yd22t, p['d22'])

    u1 = _unpool_concat(yd22, nd22, y13, n13, bt=4)
    yd11, yd11t = _cheb_block(u1, p['d11']['W'], p['d11']['b'], bt=4)
    nd11 = _stats(yd11t, p['d11'])
    yd12, yd12t = _cheb_block(yd11, p['d12']['W'], p['d12']['b'], norm=nd11, bt=4)
    nd12 = _stats(yd12t, p['d12'])

    return _cheb_block(yd12, p['d13']['W'], p['d13']['b'], norm=nd12, bt=2,
                       emit_t=False)


# speedup vs baseline: 1.0258x; 1.0258x over previous
import functools

import jax
import jax.numpy as jnp
from jax.experimental import pallas as pl


_EPS = 1e-5
_OFFS = tuple(range(1, 11)) + tuple(-k for k in range(1, 11))


def _lap_seq(z, V):
    zp = jnp.concatenate([z[:, V - 10:, :], z, z[:, :10, :]], axis=1)
    acc = jnp.zeros_like(z)
    w = -1.0 / 20.0
    for off in _OFFS:
        acc = acc + zp[:, 10 + off:10 + off + V, :] * w
    return acc


def _mm(z, wk):
    m = z.shape[0] * z.shape[1]
    return jax.lax.dot_general(
        z.reshape(m, z.shape[2]), wk, (((1,), (0,)), ((), ())),
        preferred_element_type=jnp.float32,
        precision=jax.lax.Precision.DEFAULT).reshape(
            z.shape[0], z.shape[1], -1)


def _affine_relu(y, m, d, g, be):
    return jnp.maximum((y - m) / d * g + be, 0.0)


def _cheb_body(y_ref, m_ref, d_ref, g_ref, be_ref, w_ref, b_ref, out_ref,
               outT_ref, *, act, V):
    y = y_ref[...]
    if act:
        x = _affine_relu(y, m_ref[0], d_ref[0], g_ref[0], be_ref[0])
    else:
        x = y
    t1 = _lap_seq(x, V)
    t2 = 2.0 * _lap_seq(t1, V) - x
    out_ref[...] = _mm(x, w_ref[0])
    out_ref[...] += _mm(t1, w_ref[1])
    out_ref[...] += _mm(t2, w_ref[2])
    out_ref[...] += b_ref[0]
    if outT_ref is not None:
        outT_ref[...] = jnp.swapaxes(out_ref[...], 1, 2)


def _cheb_block(y_in, W, b, norm=None, bt=2, emit_t=True):
    B, V, Cin = y_in.shape
    K, _, Cout = W.shape
    act = norm is not None
    if norm is None:
        z = jnp.zeros((1, Cin), jnp.float32)
        norm = (z, z + 1.0, z + 1.0, z)
    m, dd, g, be = norm
    if emit_t:
        body = functools.partial(_cheb_body, act=act, V=V)
        out_specs = [pl.BlockSpec((bt, V, Cout), lambda i: (i, 0, 0)),
                     pl.BlockSpec((bt, Cout, V), lambda i: (i, 0, 0))]
        out_shape = [jax.ShapeDtypeStruct((B, V, Cout), jnp.float32),
                     jax.ShapeDtypeStruct((B, Cout, V), jnp.float32)]
    else:
        def body(y_ref, m_ref, d_ref, g_ref, be_ref, w_ref, b_ref, out_ref):
            _cheb_body(y_ref, m_ref, d_ref, g_ref, be_ref, w_ref, b_ref,
                       out_ref, None, act=act, V=V)
        out_specs = pl.BlockSpec((bt, V, Cout), lambda i: (i, 0, 0))
        out_shape = jax.ShapeDtypeStruct((B, V, Cout), jnp.float32)
    vec = pl.BlockSpec((1, Cin), lambda i: (0, 0))
    res = pl.pallas_call(
        body,
        grid=(B // bt,),
        in_specs=[
            pl.BlockSpec((bt, V, Cin), lambda i: (i, 0, 0)),
            vec, vec, vec, vec,
            pl.BlockSpec((K, Cin, Cout), lambda i: (0, 0, 0)),
            pl.BlockSpec((1, Cout), lambda i: (0, 0)),
        ],
        out_specs=out_specs,
        out_shape=out_shape,
    )(y_in, m, dd, g, be, W, b.reshape(1, Cout))
    return res


def _stats(yT, pr):
    yv = jnp.transpose(yT, (0, 2, 1))
    mean = jnp.mean(yv, axis=(0, 1))
    den = jnp.sqrt(jnp.var(yv, axis=(0, 1)) + _EPS)
    return (mean.reshape(1, -1), den.reshape(1, -1),
            pr['g'].reshape(1, -1), pr['be'].reshape(1, -1))


def _pool_body(y_ref, m_ref, d_ref, g_ref, be_ref, out_ref, *, C):
    y = y_ref[...]
    xs = [_affine_relu(y[:, :, j * C:(j + 1) * C], m_ref[0], d_ref[0],
                       g_ref[0], be_ref[0]) for j in range(4)]
    out_ref[...] = jnp.maximum(jnp.maximum(xs[0], xs[1]),
                               jnp.maximum(xs[2], xs[3]))


def _pool(y, norm, bt=4):
    B, V, C = y.shape
    G = V // 4
    yg = y.reshape(B, G, 4 * C)
    m, dd, g, be = norm
    body = functools.partial(_pool_body, C=C)
    vec = pl.BlockSpec((1, C), lambda i: (0, 0))
    return pl.pallas_call(
        body,
        grid=(B // bt,),
        in_specs=[pl.BlockSpec((bt, G, 4 * C), lambda i: (i, 0, 0)),
                  vec, vec, vec, vec],
        out_specs=pl.BlockSpec((bt, G, C), lambda i: (i, 0, 0)),
        out_shape=jax.ShapeDtypeStruct((B, G, C), jnp.float32),
    )(yg, m, dd, g, be)


def _unpool_body(yl_ref, ml_ref, dl_ref, gl_ref, bl_ref,
                 ys_ref, ms_ref, ds_ref, gs_ref, bs_ref, out_ref, *, C):
    xl = _affine_relu(yl_ref[...], ml_ref[0], dl_ref[0], gl_ref[0], bl_ref[0])
    ys = ys_ref[...]
    xs = [_affine_relu(ys[:, :, j * C:(j + 1) * C], ms_ref[0], ds_ref[0],
                       gs_ref[0], bs_ref[0]) for j in range(4)]
    m = jnp.maximum(jnp.maximum(xs[0], xs[1]), jnp.maximum(xs[2], xs[3]))
    taken = jnp.zeros(xs[0].shape, jnp.bool_)
    for j in range(4):
        e = jnp.logical_and(xs[j] == m, jnp.logical_not(taken))
        taken = jnp.logical_or(taken, e)
        off = j * 2 * C
        out_ref[:, :, off:off + C] = jnp.where(e, xl, 0.0)
        out_ref[:, :, off + C:off + 2 * C] = xs[j]


def _unpool_concat(y_low, norm_low, y_skip, norm_skip, bt=4):
    B, G, C = y_low.shape
    ysg = y_skip.reshape(B, G, 4 * C)
    body = functools.partial(_unpool_body, C=C)
    vec = pl.BlockSpec((1, C), lambda i: (0, 0))
    out = pl.pallas_call(
        body,
        grid=(B // bt,),
        in_specs=[pl.BlockSpec((bt, G, C), lambda i: (i, 0, 0)),
                  vec, vec, vec, vec,
                  pl.BlockSpec((bt, G, 4 * C), lambda i: (i, 0, 0)),
                  vec, vec, vec, vec],
        out_specs=pl.BlockSpec((bt, G, 8 * C), lambda i: (i, 0, 0)),
        out_shape=jax.ShapeDtypeStruct((B, G, 8 * C), jnp.float32),
    )(y_low, *norm_low, ysg, *norm_skip)
    return out.reshape(B, 4 * G, 2 * C)


def kernel(x, params):
    p = params

    y11, y11t = _cheb_block(x, p['c11']['W'], p['c11']['b'], bt=2)
    n11 = _stats(y11t, p['c11'])
    y12, y12t = _cheb_block(y11, p['c12']['W'], p['c12']['b'], norm=n11, bt=2)
    n12 = _stats(y12t, p['c12'])
    y13, y13t = _cheb_block(y12, p['c13']['W'], p['c13']['b'], norm=n12, bt=2)
    n13 = _stats(y13t, p['c13'])

    x2in = _pool(y13, n13, bt=4)
    y21, y21t = _cheb_block(x2in, p['c21']['W'], p['c21']['b'], bt=4)
    n21 = _stats(y21t, p['c21'])
    y22, y22t = _cheb_block(y21, p['c22']['W'], p['c22']['b'], norm=n21, bt=4)
    n22 = _stats(y22t, p['c22'])
    y23, y23t = _cheb_block(y22, p['c23']['W'], p['c23']['b'], norm=n22, bt=4)
    n23 = _stats(y23t, p['c23'])

    x3in = _pool(y23, n23, bt=8)
    y31, y31t = _cheb_block(x3in, p['c31']['W'], p['c31']['b'], bt=8)
    n31 = _stats(y31t, p['c31'])
    y32, y32t = _cheb_block(y31, p['c32']['W'], p['c32']['b'], norm=n31, bt=8)
    n32 = _stats(y32t, p['c32'])
    y33, y33t = _cheb_block(y32, p['c33']['W'], p['c33']['b'], norm=n32, bt=8)
    n33 = _stats(y33t, p['c33'])

    u2 = _unpool_concat(y33, n33, y23, n23, bt=8)
    yd21, yd21t = _cheb_block(u2, p['d21']['W'], p['d21']['b'], bt=4)
    nd21 = _stats(yd21t, p['d21'])
    yd22, yd22t = _cheb_block(yd21, p['d22']['W'], p['d22']['b'], norm=nd21, bt=4)
    nd22 = _stats(yd22t, p['d22'])

    u1 = _unpool_concat(yd22, nd22, y13, n13, bt=4)
    yd11, yd11t = _cheb_block(u1, p['d11']['W'], p['d11']['b'], bt=2)
    nd11 = _stats(yd11t, p['d11'])
    yd12, yd12t = _cheb_block(yd11, p['d12']['W'], p['d12']['b'], norm=nd11, bt=2)
    nd12 = _stats(yd12t, p['d12'])

    return _cheb_block(yd12, p['d13']['W'], p['d13']['b'], norm=nd12, bt=2,
                       emit_t=False)
